# Initial kernel scaffold; baseline (speedup 1.0000x reference)
import jax
import jax.numpy as jnp
from jax.experimental import pallas as pl


def kernel(x, it, enc_W1, enc_b1, mu_W, mu_b, lv_W, lv_b, graph_G, embed_W, embed_b, gcn1_W, gcn1_b, gcn2_W, gcn2_b):
    raise NotImplementedError("write your pallas kernel here")


# speedup vs baseline: 1.6155x; 1.6155x over previous
import jax
import jax.numpy as jnp
from jax.experimental import pallas as pl
from jax.experimental.pallas import tpu as pltpu

_THRESHOLD = 0.1


def _fused(it_ref, x_ref, eps_ref, encW1_ref, encb1_ref, muW_ref, mub_ref,
           lvW_ref, lvb_ref, G_ref, w0_ref, Er_ref, embb_ref, g1W_ref,
           g1b_ref, g2w_ref, g2b_ref,
           xhat_ref, adj_ref, mu_ref, lv_ref):
    bt, n = x_ref.shape
    c1 = g1W_ref.shape[1]

    x = x_ref[...]
    h = jnp.maximum(
        jnp.dot(x, encW1_ref[...], preferred_element_type=jnp.float32)
        + encb1_ref[...], 0.0)
    mu = jnp.dot(h, muW_ref[...], preferred_element_type=jnp.float32) + mub_ref[...]
    lv = jnp.dot(h, lvW_ref[...], preferred_element_type=jnp.float32) + lvb_ref[...]
    mu_ref[...] = mu
    lv_ref[...] = lv
    z = mu + eps_ref[...] * jnp.exp(0.5 * lv)

    g = jax.nn.sigmoid(G_ref[...])
    rows = jax.lax.broadcasted_iota(jnp.int32, (n, n), 0)
    cols = jax.lax.broadcasted_iota(jnp.int32, (n, n), 1)
    g = jnp.where(rows == cols, 1.0, g)
    it = it_ref[0]
    g = jnp.where(jnp.logical_and(it > 50, g <= _THRESHOLD), 0.0, g)
    adj_ref[...] = g
    deg_in = jnp.clip(jnp.sum(g, axis=0), 1e-12, None)
    deg_out = jnp.clip(jnp.sum(g, axis=1), 1e-12, None)
    adjn = g * jax.lax.rsqrt(deg_out)[:, None] * jax.lax.rsqrt(deg_in)[None, :]

    emb = Er_ref[...] + embb_ref[...]
    hh = jnp.tanh(z[:, :, None] * w0_ref[...][None, :, :] + emb[None, :, :])

    p = jnp.dot(hh.reshape(bt * n, n), g1W_ref[...],
                preferred_element_type=jnp.float32)
    pt = jnp.swapaxes(p.reshape(bt, n, c1), 1, 2)
    s = jnp.dot(pt.reshape(bt * c1, n), adjn,
                preferred_element_type=jnp.float32)
    h1 = jnp.maximum(
        jnp.swapaxes(s.reshape(bt, c1, n), 1, 2) + g1b_ref[...][None, :, :], 0.0)

    v = jnp.sum(h1 * g2w_ref[...][None, :, :], axis=-1)
    xhat_ref[...] = jnp.dot(v, adjn, preferred_element_type=jnp.float32) \
        + g2b_ref[...]


def kernel(x, it, enc_W1, enc_b1, mu_W, mu_b, lv_W, lv_b, graph_G, embed_W,
           embed_b, gcn1_W, gcn1_b, gcn2_W, gcn2_b):
    b_size, n = x.shape
    e_dim = enc_W1.shape[1]
    c1 = gcn1_W.shape[1]
    bt = 128
    grid = b_size // bt

    eps = jax.random.normal(jax.random.key(42), (b_size, n), dtype=jnp.float32)
    it_arr = jnp.asarray(it, jnp.int32).reshape((1,))
    w0 = embed_W[0:1, :]
    e_rest = embed_W[1:, :]

    row_spec = lambda shape: pl.BlockSpec(shape, lambda i: (i, 0))
    fix_spec = lambda shape: pl.BlockSpec(shape, lambda i: (0, 0))

    out_shapes = (
        jax.ShapeDtypeStruct((b_size, n), jnp.float32),
        jax.ShapeDtypeStruct((n, n), jnp.float32),
        jax.ShapeDtypeStruct((b_size, n), jnp.float32),
        jax.ShapeDtypeStruct((b_size, n), jnp.float32),
    )
    in_specs = [
        pl.BlockSpec(memory_space=pltpu.SMEM),
        row_spec((bt, n)),
        row_spec((bt, n)),
        fix_spec((n, e_dim)),
        fix_spec((1, e_dim)),
        fix_spec((e_dim, n)),
        fix_spec((1, n)),
        fix_spec((e_dim, n)),
        fix_spec((1, n)),
        fix_spec((n, n)),
        fix_spec((1, n)),
        fix_spec((n, n)),
        fix_spec((1, n)),
        fix_spec((n, c1)),
        fix_spec((1, c1)),
        fix_spec((1, c1)),
        fix_spec((1, 1)),
    ]
    out_specs = (
        row_spec((bt, n)),
        fix_spec((n, n)),
        row_spec((bt, n)),
        row_spec((bt, n)),
    )
    x_hat, adj, mu, lv = pl.pallas_call(
        _fused,
        grid=(grid,),
        in_specs=in_specs,
        out_specs=out_specs,
        out_shape=out_shapes,
        compiler_params=pltpu.CompilerParams(
            dimension_semantics=("arbitrary",)),
    )(it_arr, x, eps, enc_W1, enc_b1.reshape(1, e_dim), mu_W,
      mu_b.reshape(1, n), lv_W, lv_b.reshape(1, n), graph_G, w0, e_rest,
      embed_b.reshape(1, n), gcn1_W, gcn1_b.reshape(1, c1),
      gcn2_W.reshape(1, c1), gcn2_b.reshape(1, 1))
    return (x_hat, adj, mu, lv)


# speedup vs baseline: 1.9918x; 1.2329x over previous
import jax
import jax.numpy as jnp
from jax.experimental import pallas as pl
from jax.experimental.pallas import tpu as pltpu

_THRESHOLD = 0.1


def _fused(it_ref, x_ref, eps_ref, encW1_ref, encb1_ref, muW_ref, mub_ref,
           lvW_ref, lvb_ref, G_ref, w0_ref, Er_ref, embb_ref, g1W_ref,
           g1bcol_ref, w2blk_ref, g2b_ref,
           xhat_ref, adj_ref, mu_ref, lv_ref):
    bt, n = x_ref.shape
    c1 = g1W_ref.shape[1]

    x = x_ref[...]
    h = jnp.maximum(
        jnp.dot(x, encW1_ref[...], preferred_element_type=jnp.float32)
        + encb1_ref[...], 0.0)
    mu = jnp.dot(h, muW_ref[...], preferred_element_type=jnp.float32) + mub_ref[...]
    lv = jnp.dot(h, lvW_ref[...], preferred_element_type=jnp.float32) + lvb_ref[...]
    mu_ref[...] = mu
    lv_ref[...] = lv
    z = mu + eps_ref[...] * jnp.exp(0.5 * lv)

    g = jax.nn.sigmoid(G_ref[...])
    rows = jax.lax.broadcasted_iota(jnp.int32, (n, n), 0)
    cols = jax.lax.broadcasted_iota(jnp.int32, (n, n), 1)
    g = jnp.where(rows == cols, 1.0, g)
    it = it_ref[0]
    g = jnp.where(jnp.logical_and(it > 50, g <= _THRESHOLD), 0.0, g)
    adj_ref[...] = g
    deg_in = jnp.clip(jnp.sum(g, axis=0), 1e-12, None)
    deg_out = jnp.clip(jnp.sum(g, axis=1), 1e-12, None)
    adjn = g * jax.lax.rsqrt(deg_out)[:, None] * jax.lax.rsqrt(deg_in)[None, :]

    emb = Er_ref[...] + embb_ref[...]
    hh = jnp.tanh(z[:, :, None] * w0_ref[...][None, :, :] + emb[None, :, :])

    p = jnp.dot(hh.reshape(bt * n, n), g1W_ref[...],
                preferred_element_type=jnp.float32)
    pt = jnp.swapaxes(p.reshape(bt, n, c1), 1, 2)
    s = jnp.dot(pt.reshape(bt * c1, n), adjn,
                preferred_element_type=jnp.float32)
    q = jnp.maximum(s.reshape(bt, c1, n) + g1bcol_ref[...][None, :, :], 0.0)

    v = jnp.dot(w2blk_ref[...], q.reshape(bt * c1, n),
                preferred_element_type=jnp.float32)
    xhat_ref[...] = jnp.dot(v, adjn, preferred_element_type=jnp.float32) \
        + g2b_ref[...]


def kernel(x, it, enc_W1, enc_b1, mu_W, mu_b, lv_W, lv_b, graph_G, embed_W,
           embed_b, gcn1_W, gcn1_b, gcn2_W, gcn2_b):
    b_size, n = x.shape
    e_dim = enc_W1.shape[1]
    c1 = gcn1_W.shape[1]
    bt = 128
    grid = b_size // bt

    eps = jax.random.normal(jax.random.key(42), (b_size, n), dtype=jnp.float32)
    it_arr = jnp.asarray(it, jnp.int32).reshape((1,))
    w0 = embed_W[0:1, :]
    e_rest = embed_W[1:, :]
    w2blk = jnp.kron(jnp.eye(bt, dtype=jnp.float32), gcn2_W.reshape(1, c1))

    row_spec = lambda shape: pl.BlockSpec(shape, lambda i: (i, 0))
    fix_spec = lambda shape: pl.BlockSpec(shape, lambda i: (0, 0))

    out_shapes = (
        jax.ShapeDtypeStruct((b_size, n), jnp.float32),
        jax.ShapeDtypeStruct((n, n), jnp.float32),
        jax.ShapeDtypeStruct((b_size, n), jnp.float32),
        jax.ShapeDtypeStruct((b_size, n), jnp.float32),
    )
    in_specs = [
        pl.BlockSpec(memory_space=pltpu.SMEM),
        row_spec((bt, n)),
        row_spec((bt, n)),
        fix_spec((n, e_dim)),
        fix_spec((1, e_dim)),
        fix_spec((e_dim, n)),
        fix_spec((1, n)),
        fix_spec((e_dim, n)),
        fix_spec((1, n)),
        fix_spec((n, n)),
        fix_spec((1, n)),
        fix_spec((n, n)),
        fix_spec((1, n)),
        fix_spec((n, c1)),
        fix_spec((c1, 1)),
        fix_spec((bt, bt * c1)),
        fix_spec((1, 1)),
    ]
    out_specs = (
        row_spec((bt, n)),
        fix_spec((n, n)),
        row_spec((bt, n)),
        row_spec((bt, n)),
    )
    x_hat, adj, mu, lv = pl.pallas_call(
        _fused,
        grid=(grid,),
        in_specs=in_specs,
        out_specs=out_specs,
        out_shape=out_shapes,
        compiler_params=pltpu.CompilerParams(
            dimension_semantics=("arbitrary",)),
    )(it_arr, x, eps, enc_W1, enc_b1.reshape(1, e_dim), mu_W,
      mu_b.reshape(1, n), lv_W, lv_b.reshape(1, n), graph_G, w0, e_rest,
      embed_b.reshape(1, n), gcn1_W, gcn1_b.reshape(c1, 1),
      w2blk, gcn2_b.reshape(1, 1))
    return (x_hat, adj, mu, lv)
